# baseline (speedup 1.0000x reference)
import functools

import jax
import jax.numpy as jnp
from jax import lax
from jax.experimental import pallas as pl
from jax.experimental.pallas import tpu as pltpu
from jax.experimental.pallas import tpu_sc as plsc

N = 10000
NP = 10240
C = 16
NC = 2
NS = 16
NW = NC * NS
SUB = 128
NSUB = 40
EPT = SUB * NSUB
E_PAD = NW * EPT
RPT = NP // NS
ZR = 160
BLK = 1024
G = NP // BLK
NB = 4


def _sc_mesh():
    return plsc.VectorSubcoreMesh(
        core_axis_name="c", subcore_axis_name="s",
        num_cores=NC, num_subcores=NS)


def _unpack_ids(pk_v, src_v, dst_v):
    def ubody(jj, _):
        for g in range(SUB // 16):
            sl = pl.ds(g * 16, 16)
            v = pk_v[pl.ds(jj * SUB + g * 16, 16)]
            src_v[jj, sl] = lax.shift_right_logical(v, 14)
            dst_v[jj, sl] = lax.bitwise_and(v, 16383)
        return 0
    lax.fori_loop(0, NSUB, ubody, 0)


def _sc_segment_mm(tables, pkb, ewb, F):
    grp = F // 16
    NT = len(tables)

    @functools.partial(
        pl.kernel,
        out_type=[jax.ShapeDtypeStruct((NC, NP, F), jnp.float32)
                  for _ in range(NT)],
        mesh=_sc_mesh(),
        scratch_types=[
            pltpu.VMEM((EPT,), jnp.int32),
            pltpu.VMEM((NSUB, SUB), jnp.int32),
            pltpu.VMEM((NSUB, SUB), jnp.int32),
            pltpu.VMEM((EPT,), jnp.float32),
        ] + [pltpu.VMEM((SUB, F), jnp.float32) for _ in range(2 * NB)] + [
            pltpu.VMEM((ZR, F), jnp.float32),
            pltpu.VMEM_SHARED((NP, F), jnp.float32),
            pltpu.VMEM_SHARED((NP, F), jnp.float32),
        ] + [pltpu.SemaphoreType.DMA for _ in range(2 * NB)],
        compiler_params=pltpu.CompilerParams(use_tc_tiling_on_sc=False),
    )
    def k(*args):
        tables_h = args[:NT]
        pk_h, ew_h = args[NT:NT + 2]
        outs_h = args[NT + 2:2 * NT + 2]
        rest = args[2 * NT + 2:]
        pk_v, src_v, dst_v, ew_v = rest[:4]
        bufs = rest[4:4 + NB]
        sbufs = rest[4 + NB:4 + 2 * NB]
        zrow_v, agg_sh, table_g = rest[4 + 2 * NB:7 + 2 * NB]
        sems = rest[7 + 2 * NB:]
        gsem = sems[:NB]
        ssem = sems[NB:]
        c = lax.axis_index("c")
        s = lax.axis_index("s")
        tid = c * NS + s
        pltpu.sync_copy(pk_h.at[pl.ds(tid * EPT, EPT)], pk_v)
        pltpu.sync_copy(ew_h.at[pl.ds(tid * EPT, EPT)], ew_v)
        _unpack_ids(pk_v, src_v, dst_v)
        base = s * RPT

        def gstart(u, jj):
            pltpu.async_copy(table_g.at[src_v.at[jj]], bufs[u], gsem[u])

        def gwait(u):
            pltpu.make_async_copy(table_g.at[src_v.at[0]], bufs[u],
                                  gsem[u]).wait()

        def sstart(u, jj):
            pltpu.async_copy(sbufs[u], agg_sh.at[dst_v.at[jj]], ssem[u],
                             add=True)

        def swait(u):
            pltpu.make_async_copy(sbufs[u], agg_sh.at[dst_v.at[0]],
                                  ssem[u]).wait()

        def scale(u, jj):
            def gbody(g, _):
                wv = ew_v[pl.ds(jj * SUB + g * 16, 16)]
                for kk in range(16):
                    w = wv[kk]
                    e = g * 16 + kk
                    for j in range(grp):
                        sl = pl.ds(j * 16, 16)
                        sbufs[u][e, sl] = bufs[u][e, sl] * w
                return 0
            lax.fori_loop(0, SUB // 16, gbody, 0)

        zv = jnp.zeros((16,), jnp.float32)

        def zbody(i, _):
            for j in range(grp):
                zrow_v[i, pl.ds(j * 16, 16)] = zv
            return 0
        lax.fori_loop(0, ZR, zbody, 0)

        for table_h, out_h in zip(tables_h, outs_h):
            pltpu.sync_copy(table_h.at[pl.ds(base, RPT)],
                            table_g.at[pl.ds(base, RPT)])
            for kk in range(RPT // ZR):
                pltpu.sync_copy(zrow_v, agg_sh.at[pl.ds(base + kk * ZR, ZR)])
            plsc.subcore_barrier()
            gstart(0, 0)
            gstart(1, 1)

            def body(gi, _):
                for u in range(NB):
                    jj = gi * NB + u
                    gwait(u)
                    nu = (u + 2) % NB

                    @pl.when(jj + 2 < NSUB)
                    def _():
                        gstart(nu, jj + 2)

                    @pl.when(jj >= NB)
                    def _():
                        swait(u)

                    scale(u, jj)
                    sstart(u, jj)
                return 0
            lax.fori_loop(0, NSUB // NB, body, 0)
            for u in range(NB):
                swait(u)
            plsc.subcore_barrier()

            pltpu.sync_copy(agg_sh.at[pl.ds(base, RPT)],
                            out_h.at[c, pl.ds(base, RPT)])

    return k(*tables, pkb, ewb)


def _sc_mincut_terms(s_soft, pkb, ewb):
    @functools.partial(
        pl.kernel,
        out_type=jax.ShapeDtypeStruct((NW, 2, 16), jnp.float32),
        mesh=_sc_mesh(),
        scratch_types=[
            pltpu.VMEM((EPT,), jnp.int32),
            pltpu.VMEM((NSUB, SUB), jnp.int32),
            pltpu.VMEM((NSUB, SUB), jnp.int32),
            pltpu.VMEM((EPT,), jnp.float32),
            pltpu.VMEM((2, SUB, 16), jnp.float32),
            pltpu.VMEM((2, SUB, 16), jnp.float32),
            pltpu.VMEM((2, 16), jnp.float32),
            pltpu.VMEM_SHARED((NP, 16), jnp.float32),
            pltpu.SemaphoreType.DMA,
            pltpu.SemaphoreType.DMA,
        ],
        compiler_params=pltpu.CompilerParams(use_tc_tiling_on_sc=False),
    )
    def k(s_h, pk_h, ew_h, out_h,
          pk_v, src_v, dst_v, ew_v, srows_v, drows_v, obuf_v, s_sh,
          sem1, sem2):
        c = lax.axis_index("c")
        s = lax.axis_index("s")
        tid = c * NS + s
        pltpu.sync_copy(pk_h.at[pl.ds(tid * EPT, EPT)], pk_v)
        pltpu.sync_copy(ew_h.at[pl.ds(tid * EPT, EPT)], ew_v)
        _unpack_ids(pk_v, src_v, dst_v)
        base = s * RPT
        pltpu.sync_copy(s_h.at[pl.ds(base, RPT)], s_sh.at[pl.ds(base, RPT)])
        plsc.subcore_barrier()

        def lstart(u, jj):
            pltpu.async_copy(s_sh.at[src_v.at[jj]], srows_v.at[u], sem1)
            pltpu.async_copy(s_sh.at[dst_v.at[jj]], drows_v.at[u], sem2)

        def lwait(u):
            pltpu.make_async_copy(s_sh.at[src_v.at[0]], srows_v.at[u],
                                  sem1).wait()
            pltpu.make_async_copy(s_sh.at[dst_v.at[0]], drows_v.at[u],
                                  sem2).wait()

        lstart(0, 0)

        def jbody(gi, carry):
            for u in range(2):
                jj = gi * 2 + u
                lwait(u)

                @pl.when(jj + 1 < NSUB)
                def _():
                    lstart(1 - u, jj + 1)

                def gbody(g, cr):
                    an, ad = cr
                    wv = ew_v[pl.ds(jj * SUB + g * 16, 16)]
                    for kk in range(16):
                        w = wv[kk]
                        e = g * 16 + kk
                        s1 = srows_v[u, e, :]
                        s2 = drows_v[u, e, :]
                        an = an + (s1 * s2) * w
                        ad = ad + (s1 * s1) * w
                    return (an, ad)
                carry = lax.fori_loop(0, SUB // 16, gbody, carry)
            return carry

        z = jnp.zeros((16,), jnp.float32)
        accn, accd = lax.fori_loop(0, NSUB // 2, jbody, (z, z))
        obuf_v[0, :] = accn
        obuf_v[1, :] = accd
        pltpu.sync_copy(obuf_v, out_h.at[tid])

    return k(s_soft, pkb, ewb)


def _tc_lin1(x, W_rel1, W_root1, b1):
    def body(x_ref, wr_ref, wo_ref, b_ref, ya_ref, yb_ref, r_ref):
        xb = x_ref[...]
        y = jnp.dot(xb, wr_ref[...], preferred_element_type=jnp.float32)
        ya_ref[...] = y[:, :32]
        yb_ref[...] = y[:, 32:]
        r_ref[...] = (jnp.dot(xb, wo_ref[...], preferred_element_type=jnp.float32)
                      + b_ref[...])
    return pl.pallas_call(
        body,
        grid=(G,),
        in_specs=[
            pl.BlockSpec((BLK, 128), lambda i: (i, 0)),
            pl.BlockSpec((128, 64), lambda i: (0, 0)),
            pl.BlockSpec((128, 64), lambda i: (0, 0)),
            pl.BlockSpec((1, 64), lambda i: (0, 0)),
        ],
        out_specs=[
            pl.BlockSpec((BLK, 32), lambda i: (i, 0)),
            pl.BlockSpec((BLK, 32), lambda i: (i, 0)),
            pl.BlockSpec((BLK, 64), lambda i: (i, 0)),
        ],
        out_shape=[
            jax.ShapeDtypeStruct((NP, 32), jnp.float32),
            jax.ShapeDtypeStruct((NP, 32), jnp.float32),
            jax.ShapeDtypeStruct((NP, 64), jnp.float32),
        ],
    )(x, W_rel1, W_root1, b1.reshape(1, 64))


def _tc_lin2(aggpa, aggpb, r1, W_rel2, W_root2, b2):
    def body(aa_ref, ab_ref, r_ref, wr_ref, wo_ref, b_ref, y2_ref, r2_ref):
        agg = jnp.concatenate(
            [aa_ref[0] + aa_ref[1], ab_ref[0] + ab_ref[1]], axis=1)
        h1 = jnp.maximum(agg + r_ref[...], 0.0)
        y2_ref[...] = jnp.dot(h1, wr_ref[...], preferred_element_type=jnp.float32)
        r2_ref[...] = (jnp.dot(h1, wo_ref[...], preferred_element_type=jnp.float32)
                       + b_ref[...])
    return pl.pallas_call(
        body,
        grid=(G,),
        in_specs=[
            pl.BlockSpec((NC, BLK, 32), lambda i: (0, i, 0)),
            pl.BlockSpec((NC, BLK, 32), lambda i: (0, i, 0)),
            pl.BlockSpec((BLK, 64), lambda i: (i, 0)),
            pl.BlockSpec((64, 32), lambda i: (0, 0)),
            pl.BlockSpec((64, 32), lambda i: (0, 0)),
            pl.BlockSpec((1, 32), lambda i: (0, 0)),
        ],
        out_specs=[
            pl.BlockSpec((BLK, 32), lambda i: (i, 0)),
            pl.BlockSpec((BLK, 32), lambda i: (i, 0)),
        ],
        out_shape=[
            jax.ShapeDtypeStruct((NP, 32), jnp.float32),
            jax.ShapeDtypeStruct((NP, 32), jnp.float32),
        ],
    )(aggpa, aggpb, r1, W_rel2, W_root2, b2.reshape(1, 32))


def _tc_assign(aggp2, r2, W_m1, b_m1, W_m2, b_m2):
    def body(a_ref, r_ref, w1_ref, b1_ref, w2_ref, b2_ref, s_ref, ss_ref):
        h2 = jnp.maximum(a_ref[0] + a_ref[1] + r_ref[...], 0.0)
        t = jnp.maximum(
            jnp.dot(h2, w1_ref[...], preferred_element_type=jnp.float32)
            + b1_ref[...], 0.0)
        s = (jnp.dot(t, w2_ref[...], preferred_element_type=jnp.float32)
             + b2_ref[...])
        m = jnp.max(s, axis=-1, keepdims=True)
        e = jnp.exp(s - m)
        ssf = e / jnp.sum(e, axis=-1, keepdims=True)
        s_ref[...] = ssf
        row = (lax.broadcasted_iota(jnp.int32, (BLK, C), 0)
               + pl.program_id(0) * BLK)
        ssf_m = jnp.where(row < N, ssf, 0.0)
        contrib = lax.dot_general(ssf_m, ssf_m, (((0,), (0,)), ((), ())),
                                  preferred_element_type=jnp.float32)

        @pl.when(pl.program_id(0) == 0)
        def _():
            ss_ref[...] = jnp.zeros_like(ss_ref)
        ss_ref[...] += contrib

    return pl.pallas_call(
        body,
        grid=(G,),
        in_specs=[
            pl.BlockSpec((NC, BLK, 32), lambda i: (0, i, 0)),
            pl.BlockSpec((BLK, 32), lambda i: (i, 0)),
            pl.BlockSpec((32, 32), lambda i: (0, 0)),
            pl.BlockSpec((1, 32), lambda i: (0, 0)),
            pl.BlockSpec((32, C), lambda i: (0, 0)),
            pl.BlockSpec((1, C), lambda i: (0, 0)),
        ],
        out_specs=[
            pl.BlockSpec((BLK, C), lambda i: (i, 0)),
            pl.BlockSpec((C, C), lambda i: (0, 0)),
        ],
        out_shape=[
            jax.ShapeDtypeStruct((NP, C), jnp.float32),
            jax.ShapeDtypeStruct((C, C), jnp.float32),
        ],
    )(aggp2, r2, W_m1, b_m1.reshape(1, 32), W_m2, b_m2.reshape(1, C))


def _tc_losses(ss, nd):
    def body(ss_ref, nd_ref, mc_ref, ol_ref):
        nd = nd_ref[...]
        num = jnp.sum(nd[:, 0, :])
        den = jnp.sum(nd[:, 1, :])
        mc_ref[...] = jnp.reshape(-(num / den), (1, 1))
        ssm = ss_ref[...]
        ssn = jnp.sqrt(jnp.sum(ssm * ssm))
        r = lax.broadcasted_iota(jnp.int32, (C, C), 0)
        c = lax.broadcasted_iota(jnp.int32, (C, C), 1)
        eye = jnp.where(r == c, 1.0 / jnp.sqrt(jnp.float32(C)), 0.0)
        d = ssm / ssn - eye
        ol_ref[...] = jnp.reshape(jnp.sqrt(jnp.sum(d * d)), (1, 1))

    return pl.pallas_call(
        body,
        out_shape=[
            jax.ShapeDtypeStruct((1, 1), jnp.float32),
            jax.ShapeDtypeStruct((1, 1), jnp.float32),
        ],
    )(ss, nd)


def kernel(x, edge_index, edge_weight, W_rel1, W_root1, b1,
           W_rel2, W_root2, b2, W_m1, b_m1, W_m2, b_m2):
    src = edge_index[0]
    dst = edge_index[1]
    E = src.shape[0]
    pad = E_PAD - E
    pkb = jnp.pad(src * 16384 + dst, (0, pad))
    ewb = jnp.pad(edge_weight, (0, pad))

    x_p = jnp.pad(x, ((0, NP - N), (0, 0)))
    y1a, y1b, r1 = _tc_lin1(x_p, W_rel1, W_root1, b1)
    aggp1a, aggp1b = _sc_segment_mm((y1a, y1b), pkb, ewb, 32)
    y2, r2 = _tc_lin2(aggp1a, aggp1b, r1, W_rel2, W_root2, b2)
    aggp2, = _sc_segment_mm((y2,), pkb, ewb, 32)
    s_soft, ss = _tc_assign(aggp2, r2, W_m1, b_m1, W_m2, b_m2)
    nd = _sc_mincut_terms(s_soft, pkb, ewb)
    mc, ol = _tc_losses(ss, nd)
    return (s_soft[:N], mc[0, 0], ol[0, 0])


# speedup vs baseline: 1.0044x; 1.0044x over previous
import functools

import jax
import jax.numpy as jnp
from jax import lax
from jax.experimental import pallas as pl
from jax.experimental.pallas import tpu as pltpu
from jax.experimental.pallas import tpu_sc as plsc

N = 10000
NP = 10240
C = 16
NC = 2
NS = 16
NW = NC * NS
SUB = 128
NSUB = 40
EPT = SUB * NSUB
E_PAD = NW * EPT
RPT = NP // NS
ZR = 160
BLK = 1024
G = NP // BLK
NB = 4


def _sc_mesh():
    return plsc.VectorSubcoreMesh(
        core_axis_name="c", subcore_axis_name="s",
        num_cores=NC, num_subcores=NS)


def _unpack_ids(pk_v, src_v, dst_v):
    def ubody(jj, _):
        for g in range(SUB // 16):
            sl = pl.ds(g * 16, 16)
            v = pk_v[pl.ds(jj * SUB + g * 16, 16)]
            src_v[jj, sl] = lax.shift_right_logical(v, 14)
            dst_v[jj, sl] = lax.bitwise_and(v, 16383)
        return 0
    lax.fori_loop(0, NSUB, ubody, 0)


def _sc_segment_mm(tables, pkb, ewb, F):
    grp = F // 16
    NT = len(tables)

    @functools.partial(
        pl.kernel,
        out_type=[jax.ShapeDtypeStruct((NC, NP, F), jnp.float32)
                  for _ in range(NT)],
        mesh=_sc_mesh(),
        scratch_types=[
            pltpu.VMEM((EPT,), jnp.int32),
            pltpu.VMEM((NSUB, SUB), jnp.int32),
            pltpu.VMEM((NSUB, SUB), jnp.int32),
            pltpu.VMEM((EPT,), jnp.float32),
        ] + [pltpu.VMEM((SUB, F), jnp.float32) for _ in range(2 * NB)] + [
            pltpu.VMEM((ZR, F), jnp.float32),
            pltpu.VMEM_SHARED((NP, F), jnp.float32),
            pltpu.VMEM_SHARED((NP, F), jnp.float32),
        ] + [pltpu.SemaphoreType.DMA for _ in range(2 * NB)],
        compiler_params=pltpu.CompilerParams(use_tc_tiling_on_sc=False),
    )
    def k(*args):
        tables_h = args[:NT]
        pk_h, ew_h = args[NT:NT + 2]
        outs_h = args[NT + 2:2 * NT + 2]
        rest = args[2 * NT + 2:]
        pk_v, src_v, dst_v, ew_v = rest[:4]
        bufs = rest[4:4 + NB]
        sbufs = rest[4 + NB:4 + 2 * NB]
        zrow_v, agg_sh, table_g = rest[4 + 2 * NB:7 + 2 * NB]
        sems = rest[7 + 2 * NB:]
        gsem = sems[:NB]
        ssem = sems[NB:]
        c = lax.axis_index("c")
        s = lax.axis_index("s")
        tid = c * NS + s
        pltpu.sync_copy(pk_h.at[pl.ds(tid * EPT, EPT)], pk_v)
        pltpu.sync_copy(ew_h.at[pl.ds(tid * EPT, EPT)], ew_v)
        _unpack_ids(pk_v, src_v, dst_v)
        base = s * RPT

        def gstart(u, jj):
            pltpu.async_copy(table_g.at[src_v.at[jj]], bufs[u], gsem[u])

        def gwait(u):
            pltpu.make_async_copy(table_g.at[src_v.at[0]], bufs[u],
                                  gsem[u]).wait()

        def sstart(u, jj):
            pltpu.async_copy(sbufs[u], agg_sh.at[dst_v.at[jj]], ssem[u],
                             add=True)

        def swait(u):
            pltpu.make_async_copy(sbufs[u], agg_sh.at[dst_v.at[0]],
                                  ssem[u]).wait()

        def scale(u, jj):
            def gbody(g, _):
                wv = ew_v[pl.ds(jj * SUB + g * 16, 16)]
                for kk in range(16):
                    w = wv[kk]
                    e = g * 16 + kk
                    for j in range(grp):
                        sl = pl.ds(j * 16, 16)
                        sbufs[u][e, sl] = bufs[u][e, sl] * w
                return 0
            lax.fori_loop(0, SUB // 16, gbody, 0)

        zv = jnp.zeros((16,), jnp.float32)

        def zbody(i, _):
            for j in range(grp):
                zrow_v[i, pl.ds(j * 16, 16)] = zv
            return 0
        lax.fori_loop(0, ZR, zbody, 0)

        for table_h, out_h in zip(tables_h, outs_h):
            pltpu.sync_copy(table_h.at[pl.ds(base, RPT)],
                            table_g.at[pl.ds(base, RPT)])
            for kk in range(RPT // ZR):
                pltpu.sync_copy(zrow_v, agg_sh.at[pl.ds(base + kk * ZR, ZR)])
            plsc.subcore_barrier()
            gstart(0, 0)
            gstart(1, 1)

            def body(gi, _):
                for u in range(NB):
                    jj = gi * NB + u
                    gwait(u)
                    nu = (u + 2) % NB

                    @pl.when(jj + 2 < NSUB)
                    def _():
                        gstart(nu, jj + 2)

                    @pl.when(jj >= NB)
                    def _():
                        swait(u)

                    scale(u, jj)
                    sstart(u, jj)
                return 0
            lax.fori_loop(0, NSUB // NB, body, 0)
            for u in range(NB):
                swait(u)
            plsc.subcore_barrier()

            pltpu.sync_copy(agg_sh.at[pl.ds(base, RPT)],
                            out_h.at[c, pl.ds(base, RPT)])

    return k(*tables, pkb, ewb)


def _sc_mincut_terms(s_soft, pkb, ewb):
    @functools.partial(
        pl.kernel,
        out_type=jax.ShapeDtypeStruct((NW, 2, 16), jnp.float32),
        mesh=_sc_mesh(),
        scratch_types=[
            pltpu.VMEM((EPT,), jnp.int32),
            pltpu.VMEM((NSUB, SUB), jnp.int32),
            pltpu.VMEM((NSUB, SUB), jnp.int32),
            pltpu.VMEM((EPT,), jnp.float32),
            pltpu.VMEM((2, SUB, 16), jnp.float32),
            pltpu.VMEM((2, SUB, 16), jnp.float32),
            pltpu.VMEM((2, 16), jnp.float32),
            pltpu.VMEM_SHARED((NP, 16), jnp.float32),
            pltpu.SemaphoreType.DMA,
            pltpu.SemaphoreType.DMA,
        ],
        compiler_params=pltpu.CompilerParams(use_tc_tiling_on_sc=False),
    )
    def k(s_h, pk_h, ew_h, out_h,
          pk_v, src_v, dst_v, ew_v, srows_v, drows_v, obuf_v, s_sh,
          sem1, sem2):
        c = lax.axis_index("c")
        s = lax.axis_index("s")
        tid = c * NS + s
        pltpu.sync_copy(pk_h.at[pl.ds(tid * EPT, EPT)], pk_v)
        pltpu.sync_copy(ew_h.at[pl.ds(tid * EPT, EPT)], ew_v)
        _unpack_ids(pk_v, src_v, dst_v)
        base = s * RPT
        pltpu.sync_copy(s_h.at[pl.ds(base, RPT)], s_sh.at[pl.ds(base, RPT)])
        plsc.subcore_barrier()

        def lstart(u, jj):
            pltpu.async_copy(s_sh.at[src_v.at[jj]], srows_v.at[u], sem1)
            pltpu.async_copy(s_sh.at[dst_v.at[jj]], drows_v.at[u], sem2)

        def lwait(u):
            pltpu.make_async_copy(s_sh.at[src_v.at[0]], srows_v.at[u],
                                  sem1).wait()
            pltpu.make_async_copy(s_sh.at[dst_v.at[0]], drows_v.at[u],
                                  sem2).wait()

        lstart(0, 0)

        def jbody(gi, carry):
            for u in range(2):
                jj = gi * 2 + u
                lwait(u)

                @pl.when(jj + 1 < NSUB)
                def _():
                    lstart(1 - u, jj + 1)

                def gbody(g, cr):
                    an, ad = cr
                    wv = ew_v[pl.ds(jj * SUB + g * 16, 16)]
                    for kk in range(16):
                        w = wv[kk]
                        e = g * 16 + kk
                        s1 = srows_v[u, e, :]
                        s2 = drows_v[u, e, :]
                        an = an + (s1 * s2) * w
                        ad = ad + (s1 * s1) * w
                    return (an, ad)
                carry = lax.fori_loop(0, SUB // 16, gbody, carry)
            return carry

        z = jnp.zeros((16,), jnp.float32)
        accn, accd = lax.fori_loop(0, NSUB // 2, jbody, (z, z))
        obuf_v[0, :] = accn
        obuf_v[1, :] = accd
        pltpu.sync_copy(obuf_v, out_h.at[tid])

    return k(s_soft, pkb, ewb)


def _tc_lin1(x, W_rel1, W_root1, b1):
    def body(x_ref, wr_ref, wo_ref, b_ref, ya_ref, yb_ref, r_ref):
        xb = x_ref[...]
        y = jnp.dot(xb, wr_ref[...], preferred_element_type=jnp.float32)
        ya_ref[...] = y[:, :32]
        yb_ref[...] = y[:, 32:]
        r_ref[...] = (jnp.dot(xb, wo_ref[...], preferred_element_type=jnp.float32)
                      + b_ref[...])
    return pl.pallas_call(
        body,
        grid=(G,),
        in_specs=[
            pl.BlockSpec((BLK, 128), lambda i: (i, 0)),
            pl.BlockSpec((128, 64), lambda i: (0, 0)),
            pl.BlockSpec((128, 64), lambda i: (0, 0)),
            pl.BlockSpec((1, 64), lambda i: (0, 0)),
        ],
        out_specs=[
            pl.BlockSpec((BLK, 32), lambda i: (i, 0)),
            pl.BlockSpec((BLK, 32), lambda i: (i, 0)),
            pl.BlockSpec((BLK, 64), lambda i: (i, 0)),
        ],
        out_shape=[
            jax.ShapeDtypeStruct((NP, 32), jnp.float32),
            jax.ShapeDtypeStruct((NP, 32), jnp.float32),
            jax.ShapeDtypeStruct((NP, 64), jnp.float32),
        ],
    )(x, W_rel1, W_root1, b1.reshape(1, 64))


def _tc_lin2(aggpa, aggpb, r1, W_rel2, W_root2, b2):
    def body(aa_ref, ab_ref, r_ref, wr_ref, wo_ref, b_ref, y2_ref, r2_ref):
        agg = jnp.concatenate(
            [aa_ref[0] + aa_ref[1], ab_ref[0] + ab_ref[1]], axis=1)
        h1 = jnp.maximum(agg + r_ref[...], 0.0)
        y2_ref[...] = jnp.dot(h1, wr_ref[...], preferred_element_type=jnp.float32)
        r2_ref[...] = (jnp.dot(h1, wo_ref[...], preferred_element_type=jnp.float32)
                       + b_ref[...])
    return pl.pallas_call(
        body,
        grid=(G,),
        in_specs=[
            pl.BlockSpec((NC, BLK, 32), lambda i: (0, i, 0)),
            pl.BlockSpec((NC, BLK, 32), lambda i: (0, i, 0)),
            pl.BlockSpec((BLK, 64), lambda i: (i, 0)),
            pl.BlockSpec((64, 32), lambda i: (0, 0)),
            pl.BlockSpec((64, 32), lambda i: (0, 0)),
            pl.BlockSpec((1, 32), lambda i: (0, 0)),
        ],
        out_specs=[
            pl.BlockSpec((BLK, 32), lambda i: (i, 0)),
            pl.BlockSpec((BLK, 32), lambda i: (i, 0)),
        ],
        out_shape=[
            jax.ShapeDtypeStruct((NP, 32), jnp.float32),
            jax.ShapeDtypeStruct((NP, 32), jnp.float32),
        ],
    )(aggpa, aggpb, r1, W_rel2, W_root2, b2.reshape(1, 32))


def _tc_assign(aggp2, r2, W_m1, b_m1, W_m2, b_m2):
    def body(a_ref, r_ref, w1_ref, b1_ref, w2_ref, b2_ref, s_ref, ss_ref):
        h2 = jnp.maximum(a_ref[0] + a_ref[1] + r_ref[...], 0.0)
        t = jnp.maximum(
            jnp.dot(h2, w1_ref[...], preferred_element_type=jnp.float32)
            + b1_ref[...], 0.0)
        s = (jnp.dot(t, w2_ref[...], preferred_element_type=jnp.float32)
             + b2_ref[...])
        m = jnp.max(s, axis=-1, keepdims=True)
        e = jnp.exp(s - m)
        ssf = e / jnp.sum(e, axis=-1, keepdims=True)
        s_ref[...] = ssf
        row = (lax.broadcasted_iota(jnp.int32, (BLK, C), 0)
               + pl.program_id(0) * BLK)
        ssf_m = jnp.where(row < N, ssf, 0.0)
        contrib = lax.dot_general(ssf_m, ssf_m, (((0,), (0,)), ((), ())),
                                  preferred_element_type=jnp.float32)

        @pl.when(pl.program_id(0) == 0)
        def _():
            ss_ref[...] = jnp.zeros_like(ss_ref)
        ss_ref[...] += contrib

    return pl.pallas_call(
        body,
        grid=(G,),
        in_specs=[
            pl.BlockSpec((NC, BLK, 32), lambda i: (0, i, 0)),
            pl.BlockSpec((BLK, 32), lambda i: (i, 0)),
            pl.BlockSpec((32, 32), lambda i: (0, 0)),
            pl.BlockSpec((1, 32), lambda i: (0, 0)),
            pl.BlockSpec((32, C), lambda i: (0, 0)),
            pl.BlockSpec((1, C), lambda i: (0, 0)),
        ],
        out_specs=[
            pl.BlockSpec((BLK, C), lambda i: (i, 0)),
            pl.BlockSpec((C, C), lambda i: (0, 0)),
        ],
        out_shape=[
            jax.ShapeDtypeStruct((NP, C), jnp.float32),
            jax.ShapeDtypeStruct((C, C), jnp.float32),
        ],
    )(aggp2, r2, W_m1, b_m1.reshape(1, 32), W_m2, b_m2.reshape(1, C))


def _tc_losses(ss, nd):
    def body(ss_ref, nd_ref, mc_ref, ol_ref):
        nd = nd_ref[...]
        num = jnp.sum(nd[:, 0, :])
        den = jnp.sum(nd[:, 1, :])
        mc_ref[...] = jnp.reshape(-(num / den), (1, 1))
        ssm = ss_ref[...]
        ssn = jnp.sqrt(jnp.sum(ssm * ssm))
        r = lax.broadcasted_iota(jnp.int32, (C, C), 0)
        c = lax.broadcasted_iota(jnp.int32, (C, C), 1)
        eye = jnp.where(r == c, 1.0 / jnp.sqrt(jnp.float32(C)), 0.0)
        d = ssm / ssn - eye
        ol_ref[...] = jnp.reshape(jnp.sqrt(jnp.sum(d * d)), (1, 1))

    return pl.pallas_call(
        body,
        out_shape=[
            jax.ShapeDtypeStruct((1, 1), jnp.float32),
            jax.ShapeDtypeStruct((1, 1), jnp.float32),
        ],
    )(ss, nd)


def kernel(x, edge_index, edge_weight, W_rel1, W_root1, b1,
           W_rel2, W_root2, b2, W_m1, b_m1, W_m2, b_m2):
    src = edge_index[0]
    dst = edge_index[1]
    E = src.shape[0]
    pad = E_PAD - E
    pkb = jnp.pad(src * 16384 + dst, (0, pad))
    ewb = jnp.pad(edge_weight, (0, pad))

    x_p = jnp.pad(x, ((0, NP - N), (0, 0)))
    y1a, y1b, r1 = _tc_lin1(x_p, W_rel1, W_root1, b1)
    aggp1a, = _sc_segment_mm((y1a,), pkb, ewb, 32)
    aggp1b, = _sc_segment_mm((y1b,), pkb, ewb, 32)
    y2, r2 = _tc_lin2(aggp1a, aggp1b, r1, W_rel2, W_root2, b2)
    aggp2, = _sc_segment_mm((y2,), pkb, ewb, 32)
    s_soft, ss = _tc_assign(aggp2, r2, W_m1, b_m1, W_m2, b_m2)
    nd = _sc_mincut_terms(s_soft, pkb, ewb)
    mc, ol = _tc_losses(ss, nd)
    return (s_soft[:N], mc[0, 0], ol[0, 0])
